# baseline (speedup 1.0000x reference)
import jax
import jax.numpy as jnp
from jax import lax
from jax.experimental import pallas as pl
from jax.experimental.pallas import tpu as pltpu
from jax.experimental.pallas import tpu_sc as plsc

DIM = 8192
ROWS = 4096
NC, NS, L = 2, 16, 16
NW = NC * NS
RPW = ROWS // NW
BW = 128
SW = 64
MROWS = 8
NRB = RPW // MROWS
W_T = 2048
W_P = 1920
NBUF_T = 3
NBUF_P = 2
T_CHUNKS = tuple((rb * MROWS, BW + h * W_T) for rb in range(NRB) for h in range(2))
P_CHUNKS = tuple((rb * MROWS, BW + 2 * W_T + h * W_P) for rb in range(NRB) for h in range(2))


def _body(x, o, tmid, pmid, lb, rb, *sems):
    t_in = sems[:NBUF_T]
    t_out = sems[NBUF_T:2 * NBUF_T]
    p_in = sems[2 * NBUF_T:2 * NBUF_T + NBUF_P]
    p_out = sems[2 * NBUF_T + NBUF_P:2 * NBUF_T + 2 * NBUF_P]
    s_lbg, s_rbg, s_lbs, s_rbs = sems[2 * NBUF_T + 2 * NBUF_P:]
    cid = lax.axis_index("c")
    sid = lax.axis_index("s")
    wid = sid * NC + cid
    base = wid * RPW

    glb = pltpu.make_async_copy(x.at[pl.ds(base, RPW), pl.ds(0, BW)], lb, s_lbg)
    grb = pltpu.make_async_copy(
        x.at[pl.ds(base, RPW), pl.ds(DIM - BW, BW)], rb, s_rbg)
    glb.start()
    grb.start()

    def mid_slice(ref, g, w):
        r0, c0 = g
        return ref.at[pl.ds(base + r0, MROWS), pl.ds(c0, w)]

    def gmid_t(g, b):
        return pltpu.make_async_copy(mid_slice(x, g, W_T), tmid.at[b], t_in[b])

    def smid_t(g, b):
        return pltpu.make_async_copy(tmid.at[b], mid_slice(o, g, W_T), t_out[b])

    def gmid_p(g, b):
        return pltpu.make_async_copy(mid_slice(x, g, W_P), pmid.at[sid, b], p_in[b])

    def smid_p(g, b):
        return pltpu.make_async_copy(pmid.at[sid, b], mid_slice(o, g, W_P), p_out[b])

    for b in range(NBUF_T):
        gmid_t(T_CHUNKS[b], b).start()
    for b in range(NBUF_P):
        gmid_p(P_CHUNKS[b], b).start()

    glb.wait()
    grb.wait()

    def row(r, carry):
        for v in range(SW // L):
            a = lb[r, pl.ds(L * v, L)]
            b_ = rb[r, pl.ds(BW - L * (v + 1), L)]
            lb[r, pl.ds(L * v, L)] = lax.rev(b_, (0,))
            rb[r, pl.ds(BW - L * (v + 1), L)] = lax.rev(a, (0,))
        return carry

    lax.fori_loop(0, RPW, row, 0)

    pltpu.make_async_copy(lb, o.at[pl.ds(base, RPW), pl.ds(0, BW)], s_lbs).start()
    pltpu.make_async_copy(
        rb, o.at[pl.ds(base, RPW), pl.ds(DIM - BW, BW)], s_rbs).start()

    def step(chunks, i, nbuf, gfn, sfn):
        n = len(chunks)
        b = i % nbuf
        gfn(chunks[i], b).wait()
        sfn(chunks[i], b).start()
        j = i + 1
        if nbuf <= j < n:
            nb = j % nbuf
            sfn(chunks[j - nbuf], nb).wait()
            gfn(chunks[j], nb).start()

    for i in range(max(len(T_CHUNKS), len(P_CHUNKS))):
        if i < len(T_CHUNKS):
            step(T_CHUNKS, i, NBUF_T, gmid_t, smid_t)
        if i < len(P_CHUNKS):
            step(P_CHUNKS, i, NBUF_P, gmid_p, smid_p)

    for b in range(NBUF_T):
        smid_t(T_CHUNKS[len(T_CHUNKS) - NBUF_T + b], b).wait()
    for b in range(NBUF_P):
        smid_p(P_CHUNKS[len(P_CHUNKS) - NBUF_P + b], b).wait()
    pltpu.make_async_copy(lb, o.at[pl.ds(base, RPW), pl.ds(0, BW)], s_lbs).wait()
    pltpu.make_async_copy(
        rb, o.at[pl.ds(base, RPW), pl.ds(DIM - BW, BW)], s_rbs).wait()


def kernel(x0, x1, x2):
    mesh = plsc.VectorSubcoreMesh(
        core_axis_name="c", subcore_axis_name="s",
        num_cores=NC, num_subcores=NS)
    k = pl.kernel(
        _body,
        out_type=jax.ShapeDtypeStruct((ROWS, DIM), jnp.float32),
        mesh=mesh,
        scratch_types=(
            [pltpu.VMEM((NBUF_T, MROWS, W_T), jnp.float32),
             pltpu.VMEM_SHARED((NS, NBUF_P, MROWS, W_P), jnp.float32),
             pltpu.VMEM((RPW, BW), jnp.float32),
             pltpu.VMEM((RPW, BW), jnp.float32)]
            + [pltpu.SemaphoreType.DMA] * (2 * NBUF_T + 2 * NBUF_P + 4)
        ),
    )
    mixed = k(x0)
    return (mixed, x1, x2)


# speedup vs baseline: 1.0368x; 1.0368x over previous
import jax
import jax.numpy as jnp
from jax import lax
from jax.experimental import pallas as pl
from jax.experimental.pallas import tpu as pltpu
from jax.experimental.pallas import tpu_sc as plsc

DIM = 8192
ROWS = 4096
NC, NS, L = 2, 16, 16
NW = NC * NS
RPW = ROWS // NW
BW = 128
SW = 64
MIDW = (DIM - 2 * BW) // 2
MROWS = 8
NRB = RPW // MROWS
NMID = NRB * 2
NBUF = 3


def _body(x, o, pmid, lb, rb, *sems):
    m_in = sems[:NBUF]
    m_out = sems[NBUF:2 * NBUF]
    s_lbg, s_rbg, s_lbs, s_rbs = sems[2 * NBUF:]
    wid = lax.axis_index("s") * NC + lax.axis_index("c")
    sid = lax.axis_index("s")
    base = wid * RPW

    glb = pltpu.make_async_copy(x.at[pl.ds(base, RPW), pl.ds(0, BW)], lb, s_lbg)
    grb = pltpu.make_async_copy(
        x.at[pl.ds(base, RPW), pl.ds(DIM - BW, BW)], rb, s_rbg)
    glb.start()
    grb.start()

    def mid_slice(ref, g):
        r0 = base + (g // 2) * MROWS
        c0 = BW + (g % 2) * MIDW
        return ref.at[pl.ds(r0, MROWS), pl.ds(c0, MIDW)]

    def gmid(g, b):
        return pltpu.make_async_copy(mid_slice(x, g), pmid.at[sid, b], m_in[b])

    def smid(g, b):
        return pltpu.make_async_copy(pmid.at[sid, b], mid_slice(o, g), m_out[b])

    for b in range(NBUF):
        gmid(b, b).start()

    glb.wait()
    grb.wait()

    def row(r, carry):
        for v in range(SW // L):
            a = lb[r, pl.ds(L * v, L)]
            b_ = rb[r, pl.ds(BW - L * (v + 1), L)]
            lb[r, pl.ds(L * v, L)] = lax.rev(b_, (0,))
            rb[r, pl.ds(BW - L * (v + 1), L)] = lax.rev(a, (0,))
        return carry

    lax.fori_loop(0, RPW, row, 0)

    pltpu.make_async_copy(lb, o.at[pl.ds(base, RPW), pl.ds(0, BW)], s_lbs).start()
    pltpu.make_async_copy(
        rb, o.at[pl.ds(base, RPW), pl.ds(DIM - BW, BW)], s_rbs).start()

    for g in range(NMID):
        b = g % NBUF
        gmid(g, b).wait()
        smid(g, b).start()
        nxt = g + 1
        if NBUF <= nxt < NMID:
            nb = nxt % NBUF
            smid(nxt - NBUF, nb).wait()
            gmid(nxt, nb).start()

    for b in range(NBUF):
        smid(NMID - NBUF + b, b).wait()
    pltpu.make_async_copy(lb, o.at[pl.ds(base, RPW), pl.ds(0, BW)], s_lbs).wait()
    pltpu.make_async_copy(
        rb, o.at[pl.ds(base, RPW), pl.ds(DIM - BW, BW)], s_rbs).wait()


def kernel(x0, x1, x2):
    mesh = plsc.VectorSubcoreMesh(
        core_axis_name="c", subcore_axis_name="s",
        num_cores=NC, num_subcores=NS)
    k = pl.kernel(
        _body,
        out_type=jax.ShapeDtypeStruct((ROWS, DIM), jnp.float32),
        mesh=mesh,
        scratch_types=(
            [pltpu.VMEM_SHARED((NS, NBUF, MROWS, MIDW), jnp.float32),
             pltpu.VMEM((RPW, BW), jnp.float32),
             pltpu.VMEM((RPW, BW), jnp.float32)]
            + [pltpu.SemaphoreType.DMA] * (2 * NBUF + 4)
        ),
    )
    mixed = k(x0)
    return (mixed, x1, x2)
